# baseline (speedup 1.0000x reference)
import functools

import jax
import jax.numpy as jnp
from jax import lax
from jax.experimental import pallas as pl
from jax.experimental.pallas import tpu as pltpu
from jax.experimental.pallas import tpu_sc as plsc

B = 16384
V = 1000000
D = 64

NC = 2
NS = 16
NW = NC * NS
B_PER_W = B // NW
L = 16
K = 16


def _body(tab, idxq, idxa, outq, outa, idx_v, ring_v, o_v,
          sem0, sem1, sem2):
    wid = lax.axis_index("s") * NC + lax.axis_index("c")
    base = wid * B_PER_W
    iota = lax.iota(jnp.int32, L)
    NCH = B_PER_W // L
    sems = (sem0, sem1, sem2)

    def drain16(sem):
        for _ in range(L):
            pltpu.make_async_copy(tab.at[pl.ds(0, 8), :], ring_v.at[0],
                                  sem).wait()

    for st, (idx_hbm, out_hbm) in enumerate(((idxq, outq), (idxa, outa))):
        pltpu.sync_copy(idx_hbm.at[pl.ds(base, B_PER_W)],
                        idx_v.at[pl.ds(0, B_PER_W)])

        def fire16(c, grp):
            u16 = idx_v[pl.ds(c * L, L)]
            for k in range(L):
                u8 = pl.multiple_of((u16[k] >> 3) << 3, 8)
                pltpu.async_copy(tab.at[pl.ds(u8, 8), :],
                                 ring_v.at[grp * L + k], sems[grp])

        def select16(c, grp):
            u16 = idx_v[pl.ds(c * L, L)]
            rv16 = u16 & 7
            for k in range(L):
                slotv = jnp.zeros((L,), jnp.int32) + (grp * L + k)
                rv = jnp.zeros((L,), jnp.int32) + rv16[k]
                iv = jnp.zeros((L,), jnp.int32) + (c * L + k)
                for j in range(D // L):
                    cols = j * L + iota
                    x = plsc.load_gather(ring_v, [slotv, rv, cols])
                    plsc.store_scatter(o_v, [iv, cols], x)

        fire16(0, 0)
        fire16(1, 1)

        def step(c, _):
            for g in range(3):
                @pl.when((c % 3) == g)
                def _(g=g):
                    @pl.when(c + 2 < NCH)
                    def _():
                        fire16(c + 2, (g + 2) % 3)
                    drain16(sems[g])
                    select16(c, g)
            return 0

        lax.fori_loop(0, NCH, step, 0)

        pltpu.sync_copy(o_v, out_hbm.at[pl.ds(base, B_PER_W)])


@jax.jit
def _gather2(tab, idx_q, idx_a):
    run = functools.partial(
        pl.kernel,
        out_type=(
            jax.ShapeDtypeStruct((B, 2 * D), jnp.float32),
            jax.ShapeDtypeStruct((B, 2 * D), jnp.float32),
        ),
        mesh=plsc.VectorSubcoreMesh(core_axis_name="c", subcore_axis_name="s"),
        scratch_types=[
            pltpu.VMEM((B_PER_W + L,), jnp.int32),
            pltpu.VMEM((3 * L, 8, D), jnp.float32),
            pltpu.VMEM((B_PER_W, 2 * D), jnp.float32),
            pltpu.SemaphoreType.DMA,
            pltpu.SemaphoreType.DMA,
            pltpu.SemaphoreType.DMA,
        ],
        compiler_params=pltpu.CompilerParams(
            use_tc_tiling_on_sc=True, needs_layout_passes=False),
    )(_body)
    return run(tab, idx_q, idx_a)


def kernel(cat_q, num_q, cat_qu, num_qu, cat_au, num_au, emb_user):
    idx_q = cat_qu.reshape(B)
    idx_a = cat_au.reshape(B)
    q_full, a_full = _gather2(emb_user, idx_q, idx_a)
    return (q_full[:, :D], a_full[:, :D])
